# baseline (speedup 1.0000x reference)
import functools

import jax
import jax.numpy as jnp
from jax import lax
from jax.experimental import pallas as pl
from jax.experimental.pallas import tpu as pltpu
from jax.experimental.pallas import tpu_sc as plsc

NUM_CORES = 2
NUM_SUBCORES = 16
NUM_WORKERS = NUM_CORES * NUM_SUBCORES
LANES = 16

BATCH = 4096
HIST = 200
DIM = 64
TOTAL = BATCH * HIST
PER_WORKER = TOTAL // NUM_WORKERS
CHUNK = 128
CHUNKS_PER_WORKER = PER_WORKER // CHUNK
SCALE = 8.0

_mesh = plsc.VectorSubcoreMesh(core_axis_name="c", subcore_axis_name="s")


@functools.partial(
    pl.kernel,
    out_type=jax.ShapeDtypeStruct((TOTAL, DIM), jnp.float32),
    mesh=_mesh,
    scratch_types=[
        pltpu.VMEM((CHUNKS_PER_WORKER, CHUNK), jnp.int32),
        pltpu.VMEM((CHUNK, DIM), jnp.float32),
        pltpu.SemaphoreType.DMA,
    ],
    compiler_params=pltpu.CompilerParams(use_tc_tiling_on_sc=False),
)
def _embed_sc(idx_hbm, table_hbm, out_hbm, idx_v, rows_v, sem):
    wid = lax.axis_index("s") * NUM_CORES + lax.axis_index("c")
    pltpu.sync_copy(idx_hbm.at[pl.ds(wid * CHUNKS_PER_WORKER, CHUNKS_PER_WORKER)],
                    idx_v)
    out_base = wid * PER_WORKER

    def chunk_body(j, carry):
        pltpu.async_copy(table_hbm.at[idx_v.at[j]], rows_v, sem).wait()

        def row_body(r, c2):
            for cc in range(DIM // LANES):
                sl = pl.ds(cc * LANES, LANES)
                rows_v[r, sl] = rows_v[r, sl] * SCALE
            return c2

        lax.fori_loop(0, CHUNK, row_body, 0)
        pltpu.sync_copy(rows_v,
                        out_hbm.at[pl.ds(out_base + j * CHUNK, CHUNK)])
        return carry

    lax.fori_loop(0, CHUNKS_PER_WORKER, chunk_body, 0)


def kernel(x, embedding):
    idx = x.astype(jnp.int32).reshape(TOTAL // CHUNK, CHUNK)
    out = _embed_sc(idx, embedding)
    return out.reshape(BATCH, HIST, DIM)


# speedup vs baseline: 1.1017x; 1.1017x over previous
import functools

import jax
import jax.numpy as jnp
from jax import lax
from jax.experimental import pallas as pl
from jax.experimental.pallas import tpu as pltpu
from jax.experimental.pallas import tpu_sc as plsc

NUM_CORES = 2
NUM_SUBCORES = 16
NUM_WORKERS = NUM_CORES * NUM_SUBCORES
LANES = 16

BATCH = 4096
HIST = 200
DIM = 64
TOTAL = BATCH * HIST
PER_WORKER = TOTAL // NUM_WORKERS
CHUNK = 128
CHUNKS_PER_WORKER = PER_WORKER // CHUNK
SCALE = 8.0
NBUF = 4

_mesh = plsc.VectorSubcoreMesh(core_axis_name="c", subcore_axis_name="s")


@functools.partial(
    pl.kernel,
    out_type=jax.ShapeDtypeStruct((TOTAL, DIM), jnp.float32),
    mesh=_mesh,
    scratch_types=[
        pltpu.VMEM((CHUNKS_PER_WORKER, CHUNK), jnp.int32),
        pltpu.VMEM((NBUF, CHUNK, DIM), jnp.float32),
        pltpu.VMEM((NBUF, CHUNK, DIM), jnp.float32),
        pltpu.SemaphoreType.DMA((NBUF,)),
        pltpu.SemaphoreType.DMA((NBUF,)),
    ],
    compiler_params=pltpu.CompilerParams(use_tc_tiling_on_sc=False),
)
def _embed_sc(idx_hbm, table_hbm, out_hbm, idx_v, in_v, out_v, gsem, ssem):
    wid = lax.axis_index("s") * NUM_CORES + lax.axis_index("c")
    pltpu.sync_copy(idx_hbm.at[pl.ds(wid * CHUNKS_PER_WORKER, CHUNKS_PER_WORKER)],
                    idx_v)
    out_base = wid * PER_WORKER

    def start_gather(b, j):
        pltpu.async_copy(table_hbm.at[idx_v.at[j]], in_v.at[b], gsem.at[b])

    def wait_gather(b, j):
        pltpu.make_async_copy(table_hbm.at[idx_v.at[j]], in_v.at[b],
                              gsem.at[b]).wait()

    def start_store(b, j):
        pltpu.async_copy(out_v.at[b],
                         out_hbm.at[pl.ds(out_base + j * CHUNK, CHUNK)],
                         ssem.at[b])

    def wait_store(b, j):
        pltpu.make_async_copy(out_v.at[b],
                              out_hbm.at[pl.ds(out_base + j * CHUNK, CHUNK)],
                              ssem.at[b]).wait()

    def scale_chunk(b):
        def row_body(r, c2):
            for cc in range(DIM // LANES):
                sl = pl.ds(cc * LANES, LANES)
                out_v[b, r, sl] = in_v[b, r, sl] * SCALE
            return c2

        lax.fori_loop(0, CHUNK, row_body, 0, unroll=4)

    for b in range(NBUF):
        start_gather(b, b)
    for b in range(NBUF):
        wait_gather(b, b)
        scale_chunk(b)
        start_gather(b, NBUF + b)
        start_store(b, b)

    def block_body(j0, carry):
        for b in range(NBUF):
            j = j0 * NBUF + b
            wait_gather(b, j)
            wait_store(b, j - NBUF)
            scale_chunk(b)
            start_gather(b, j + NBUF)
            start_store(b, j)
        return carry

    lax.fori_loop(1, CHUNKS_PER_WORKER // NBUF - 1, block_body, 0)

    last = CHUNKS_PER_WORKER - NBUF
    for b in range(NBUF):
        j = last + b
        wait_gather(b, j)
        wait_store(b, j - NBUF)
        scale_chunk(b)
        start_store(b, j)
    for b in range(NBUF):
        wait_store(b, last + b)


def kernel(x, embedding):
    idx = x.astype(jnp.int32).reshape(TOTAL // CHUNK, CHUNK)
    out = _embed_sc(idx, embedding)
    return out.reshape(BATCH, HIST, DIM)
